# baseline (speedup 1.0000x reference)
import jax
import jax.numpy as jnp
from jax import lax
from jax.experimental import pallas as pl
from jax.experimental.pallas import tpu as pltpu
from jax.experimental.pallas import tpu_sc as plsc

L = 16
NC = 2
NS = 16
NW = NC * NS
B = 16384
D = 64
TR = 8
BPW = B // NW
CH = 16
NCHUNK = BPW // CH


def _mf_body(ut, it, ubt, ibt, utile, itile, urow, irow, out,
             utile_v, itile_v, urow_v, irow_v,
             ubuf, ibuf, ubbuf, ibbuf, out_v, sem):
    wid = lax.axis_index("s") * NC + lax.axis_index("c")
    base = wid * BPW

    pltpu.sync_copy(utile.at[pl.ds(base, BPW)], utile_v)
    pltpu.sync_copy(itile.at[pl.ds(base, BPW)], itile_v)
    pltpu.sync_copy(urow.at[pl.ds(base, BPW)], urow_v)
    pltpu.sync_copy(irow.at[pl.ds(base, BPW)], irow_v)

    lane = lax.iota(jnp.int32, L)

    def chunk(j, carry):
        cb = pl.multiple_of(j * CH, CH)
        ut_ts = utile_v[pl.ds(cb, CH)]
        it_ts = itile_v[pl.ds(cb, CH)]
        urs = urow_v[pl.ds(cb, CH)]
        irs = irow_v[pl.ds(cb, CH)]
        copies = []
        for s in range(CH):
            ut_t = ut_ts[s]
            it_t = it_ts[s]
            ur = urs[s]
            ir = irs[s]
            copies.append(pltpu.async_copy(ut.at[ut_t, ur], ubuf.at[s], sem))
            copies.append(pltpu.async_copy(it.at[it_t, ir], ibuf.at[s], sem))
            copies.append(pltpu.async_copy(ubt.at[ut_t], ubbuf.at[s], sem))
            copies.append(pltpu.async_copy(ibt.at[it_t], ibbuf.at[s], sem))
        for c in copies:
            c.wait()
        zero = jnp.zeros((L,), jnp.int32)
        acc = (plsc.load_gather(ubbuf, [lane, urs, zero])
               + plsc.load_gather(ibbuf, [lane, irs, zero]))
        for k in range(D):
            col = lax.bitwise_and(lane + k, D - 1)
            u = plsc.load_gather(ubuf, [lane, col])
            v = plsc.load_gather(ibuf, [lane, col])
            acc = acc + u * v
        out_v[pl.ds(cb, CH)] = acc
        return carry

    lax.fori_loop(0, NCHUNK, chunk, 0)
    pltpu.sync_copy(out_v, out.at[pl.ds(base, BPW)])


@jax.jit
def _mf(ut3, it3, ubt3, ibt3, utile, itile, urow, irow):
    mesh = plsc.VectorSubcoreMesh(core_axis_name="c", subcore_axis_name="s")
    kern = pl.kernel(
        _mf_body,
        mesh=mesh,
        compiler_params=pltpu.CompilerParams(needs_layout_passes=False),
        out_type=jax.ShapeDtypeStruct((B,), jnp.float32),
        scratch_types=[
            pltpu.VMEM((BPW,), jnp.int32),
            pltpu.VMEM((BPW,), jnp.int32),
            pltpu.VMEM((BPW,), jnp.int32),
            pltpu.VMEM((BPW,), jnp.int32),
            pltpu.VMEM((CH, D), jnp.float32),
            pltpu.VMEM((CH, D), jnp.float32),
            pltpu.VMEM((CH, TR, 1), jnp.float32),
            pltpu.VMEM((CH, TR, 1), jnp.float32),
            pltpu.VMEM((BPW,), jnp.float32),
            pltpu.SemaphoreType.DMA,
        ],
    )
    return kern(ut3, it3, ubt3, ibt3, utile, itile, urow, irow)


def kernel(user_table, item_table, user_bias_table, item_bias_table,
           global_bias, user_ids, item_ids):
    uid = user_ids.astype(jnp.int32)
    iid = item_ids.astype(jnp.int32)
    nu = user_table.shape[0]
    ni = item_table.shape[0]
    out = _mf(user_table.reshape(nu // TR, TR, D),
              item_table.reshape(ni // TR, TR, D),
              user_bias_table.reshape(nu // TR, TR, 1),
              item_bias_table.reshape(ni // TR, TR, 1),
              uid >> 3, iid >> 3, uid & 7, iid & 7)
    return out + global_bias[0]


# speedup vs baseline: 1.4660x; 1.4660x over previous
import jax
import jax.numpy as jnp
from jax import lax
from jax.experimental import pallas as pl
from jax.experimental.pallas import tpu as pltpu
from jax.experimental.pallas import tpu_sc as plsc

L = 16
NC = 2
NS = 16
NW = NC * NS
B = 16384
D = 64
BPW = B // NW
CH = 16
NCHUNK = BPW // CH


def _mf_body(ut, it, ubt, ibt, uid, iid, out,
             uid_v, iid_v, ubuf, ibuf, ubbuf, ibbuf, out_v, sem):
    wid = lax.axis_index("s") * NC + lax.axis_index("c")
    base = wid * BPW

    pltpu.sync_copy(uid.at[pl.ds(base, BPW)], uid_v)
    pltpu.sync_copy(iid.at[pl.ds(base, BPW)], iid_v)

    lane = lax.iota(jnp.int32, L)

    def chunk(j, carry):
        cb = pl.multiple_of(j * CH, CH)
        uids = uid_v[pl.ds(cb, CH)]
        iids = iid_v[pl.ds(cb, CH)]
        copies = []
        for s in range(CH):
            u_id = uids[s]
            i_id = iids[s]
            copies.append(pltpu.async_copy(ut.at[u_id], ubuf.at[s], sem))
            copies.append(pltpu.async_copy(it.at[i_id], ibuf.at[s], sem))
            copies.append(pltpu.async_copy(ubt.at[u_id], ubbuf.at[s], sem))
            copies.append(pltpu.async_copy(ibt.at[i_id], ibbuf.at[s], sem))
        for c in copies:
            c.wait()
        zero = jnp.zeros((L,), jnp.int32)
        acc = (plsc.load_gather(ubbuf, [lane, zero])
               + plsc.load_gather(ibbuf, [lane, zero]))
        for k in range(D):
            col = lax.bitwise_and(lane + k, D - 1)
            u = plsc.load_gather(ubuf, [lane, col])
            v = plsc.load_gather(ibuf, [lane, col])
            acc = acc + u * v
        out_v[pl.ds(cb, CH)] = acc
        return carry

    lax.fori_loop(0, NCHUNK, chunk, 0)
    pltpu.sync_copy(out_v, out.at[pl.ds(base, BPW)])


@jax.jit
def _mf(ut, it, ubt, ibt, uid, iid):
    mesh = plsc.VectorSubcoreMesh(core_axis_name="c", subcore_axis_name="s")
    kern = pl.kernel(
        _mf_body,
        mesh=mesh,
        compiler_params=pltpu.CompilerParams(needs_layout_passes=False),
        out_type=jax.ShapeDtypeStruct((B,), jnp.float32),
        scratch_types=[
            pltpu.VMEM((BPW,), jnp.int32),
            pltpu.VMEM((BPW,), jnp.int32),
            pltpu.VMEM((CH, D), jnp.float32),
            pltpu.VMEM((CH, D), jnp.float32),
            pltpu.VMEM((CH, 1), jnp.float32),
            pltpu.VMEM((CH, 1), jnp.float32),
            pltpu.VMEM((BPW,), jnp.float32),
            pltpu.SemaphoreType.DMA,
        ],
    )
    return kern(ut, it, ubt, ibt, uid, iid)


def kernel(user_table, item_table, user_bias_table, item_bias_table,
           global_bias, user_ids, item_ids):
    out = _mf(user_table, item_table, user_bias_table, item_bias_table,
              user_ids.astype(jnp.int32), item_ids.astype(jnp.int32))
    return out + global_bias[0]


# speedup vs baseline: 1.4683x; 1.0016x over previous
import jax
import jax.numpy as jnp
from jax import lax
from jax.experimental import pallas as pl
from jax.experimental.pallas import tpu as pltpu
from jax.experimental.pallas import tpu_sc as plsc

L = 16
NC = 2
NS = 16
NW = NC * NS
B = 16384
D = 64
BPW = B // NW
CH = 16
NCHUNK = BPW // CH


NSEM = 8


def _mf_body(ut, it, ubt, ibt, uid, iid, out,
             uid_v, iid_v, ubuf, ibuf, ubbuf, ibbuf, out_v, *sems):
    wid = lax.axis_index("s") * NC + lax.axis_index("c")
    base = wid * BPW

    pltpu.sync_copy(uid.at[pl.ds(base, BPW)], uid_v)
    pltpu.sync_copy(iid.at[pl.ds(base, BPW)], iid_v)

    lane = lax.iota(jnp.int32, L)

    def chunk(j, carry):
        cb = pl.multiple_of(j * CH, CH)
        uids = uid_v[pl.ds(cb, CH)]
        iids = iid_v[pl.ds(cb, CH)]
        copies = []
        for s in range(CH):
            u_id = uids[s]
            i_id = iids[s]
            copies.append(pltpu.async_copy(
                ut.at[u_id], ubuf.at[s], sems[(4 * s) % NSEM]))
            copies.append(pltpu.async_copy(
                it.at[i_id], ibuf.at[s], sems[(4 * s + 1) % NSEM]))
            copies.append(pltpu.async_copy(
                ubt.at[u_id], ubbuf.at[s], sems[(4 * s + 2) % NSEM]))
            copies.append(pltpu.async_copy(
                ibt.at[i_id], ibbuf.at[s], sems[(4 * s + 3) % NSEM]))
        for c in copies:
            c.wait()
        zero = jnp.zeros((L,), jnp.int32)
        acc = (plsc.load_gather(ubbuf, [lane, zero])
               + plsc.load_gather(ibbuf, [lane, zero]))
        for k in range(D):
            col = lax.bitwise_and(lane + k, D - 1)
            u = plsc.load_gather(ubuf, [lane, col])
            v = plsc.load_gather(ibuf, [lane, col])
            acc = acc + u * v
        out_v[pl.ds(cb, CH)] = acc
        return carry

    lax.fori_loop(0, NCHUNK, chunk, 0)
    pltpu.sync_copy(out_v, out.at[pl.ds(base, BPW)])


@jax.jit
def _mf(ut, it, ubt, ibt, uid, iid):
    mesh = plsc.VectorSubcoreMesh(core_axis_name="c", subcore_axis_name="s")
    kern = pl.kernel(
        _mf_body,
        mesh=mesh,
        compiler_params=pltpu.CompilerParams(needs_layout_passes=False),
        out_type=jax.ShapeDtypeStruct((B,), jnp.float32),
        scratch_types=[
            pltpu.VMEM((BPW,), jnp.int32),
            pltpu.VMEM((BPW,), jnp.int32),
            pltpu.VMEM((CH, D), jnp.float32),
            pltpu.VMEM((CH, D), jnp.float32),
            pltpu.VMEM((CH, 1), jnp.float32),
            pltpu.VMEM((CH, 1), jnp.float32),
            pltpu.VMEM((BPW,), jnp.float32),
        ] + [pltpu.SemaphoreType.DMA] * NSEM,
    )
    return kern(ut, it, ubt, ibt, uid, iid)


def kernel(user_table, item_table, user_bias_table, item_bias_table,
           global_bias, user_ids, item_ids):
    out = _mf(user_table, item_table, user_bias_table, item_bias_table,
              user_ids.astype(jnp.int32), item_ids.astype(jnp.int32))
    return out + global_bias[0]


# speedup vs baseline: 1.5168x; 1.0330x over previous
import functools

import jax
import jax.numpy as jnp
from jax import lax
from jax.experimental import pallas as pl
from jax.experimental.pallas import tpu as pltpu
from jax.experimental.pallas import tpu_sc as plsc

L = 16
NC = 2
NS = 16
NW = NC * NS
B = 16384
D = 64
BPW = B // NW
CH = 128
NCHUNK = BPW // CH
GRP = BPW // L

_MESH = plsc.VectorSubcoreMesh(core_axis_name="c", subcore_axis_name="s")
_PARAMS = pltpu.CompilerParams(use_tc_tiling_on_sc=False,
                               needs_layout_passes=False)


def _worker_base():
    return (lax.axis_index("s") * NC + lax.axis_index("c")) * BPW


def _rows_body(table, ids, out, ids_v, rows_v, sem):
    base = _worker_base()
    for j in range(NCHUNK):
        pltpu.sync_copy(ids.at[pl.ds(base + j * CH, CH)], ids_v.at[j])
    copies = []
    for j in range(NCHUNK):
        copies.append(pltpu.async_copy(
            table.at[ids_v.at[j]], rows_v.at[pl.ds(j * CH, CH)], sem))
    for c in copies:
        c.wait()
    pltpu.sync_copy(rows_v, out.at[pl.ds(base, BPW)])


def _bias_body(table, ids, out, ids_v, vals_v, sem):
    base = _worker_base()
    for j in range(NCHUNK):
        pltpu.sync_copy(ids.at[pl.ds(base + j * CH, CH)], ids_v.at[j])
    copies = []
    for j in range(NCHUNK):
        copies.append(pltpu.async_copy(
            table.at[ids_v.at[j]], vals_v.at[pl.ds(j * CH, CH)], sem))
    for c in copies:
        c.wait()
    pltpu.sync_copy(vals_v, out.at[pl.ds(base, BPW)])


def _combine_body(urows, irows, ub, ib, gb, out,
                  ur_v, ir_v, ub_v, ib_v, gb_v, out_v, sem):
    base = _worker_base()
    cu = pltpu.async_copy(urows.at[pl.ds(base, BPW)], ur_v, sem)
    ci = pltpu.async_copy(irows.at[pl.ds(base, BPW)], ir_v, sem)
    cub = pltpu.async_copy(ub.at[pl.ds(base, BPW)], ub_v, sem)
    cib = pltpu.async_copy(ib.at[pl.ds(base, BPW)], ib_v, sem)
    cgb = pltpu.async_copy(gb, gb_v, sem)
    cu.wait()
    ci.wait()
    cub.wait()
    cib.wait()
    cgb.wait()
    lane = lax.iota(jnp.int32, L)
    gvec = gb_v[...]

    def group(g, carry):
        gb_off = pl.multiple_of(g * L, L)
        rows = gb_off + lane
        acc = ub_v[pl.ds(gb_off, L)] + ib_v[pl.ds(gb_off, L)] + gvec
        for k in range(D):
            col = lax.bitwise_and(lane + k, D - 1)
            u = plsc.load_gather(ur_v, [rows, col])
            v = plsc.load_gather(ir_v, [rows, col])
            acc = acc + u * v
        out_v[pl.ds(gb_off, L)] = acc
        return carry

    lax.fori_loop(0, GRP, group, 0)
    pltpu.sync_copy(out_v, out.at[pl.ds(base, BPW)])


def _make_rows_kernel():
    return pl.kernel(
        _rows_body, mesh=_MESH, compiler_params=_PARAMS,
        out_type=jax.ShapeDtypeStruct((B, D), jnp.float32),
        scratch_types=[
            pltpu.VMEM((NCHUNK, CH), jnp.int32),
            pltpu.VMEM((BPW, D), jnp.float32),
            pltpu.SemaphoreType.DMA,
        ],
    )


def _make_bias_kernel():
    return pl.kernel(
        _bias_body, mesh=_MESH, compiler_params=_PARAMS,
        out_type=jax.ShapeDtypeStruct((B,), jnp.float32),
        scratch_types=[
            pltpu.VMEM((NCHUNK, CH), jnp.int32),
            pltpu.VMEM((BPW,), jnp.float32),
            pltpu.SemaphoreType.DMA,
        ],
    )


def _make_combine_kernel():
    return pl.kernel(
        _combine_body, mesh=_MESH, compiler_params=_PARAMS,
        out_type=jax.ShapeDtypeStruct((B,), jnp.float32),
        scratch_types=[
            pltpu.VMEM((BPW, D), jnp.float32),
            pltpu.VMEM((BPW, D), jnp.float32),
            pltpu.VMEM((BPW,), jnp.float32),
            pltpu.VMEM((BPW,), jnp.float32),
            pltpu.VMEM((L,), jnp.float32),
            pltpu.VMEM((BPW,), jnp.float32),
            pltpu.SemaphoreType.DMA,
        ],
    )


@jax.jit
def _mf(ut, it, ubt_flat, ibt_flat, gb16, uid, iid):
    urows = _make_rows_kernel()(ut, uid)
    irows = _make_rows_kernel()(it, iid)
    ub = _make_bias_kernel()(ubt_flat, uid)
    ib = _make_bias_kernel()(ibt_flat, iid)
    return _make_combine_kernel()(urows, irows, ub, ib, gb16)


def kernel(user_table, item_table, user_bias_table, item_bias_table,
           global_bias, user_ids, item_ids):
    return _mf(user_table, item_table,
               user_bias_table.reshape(-1), item_bias_table.reshape(-1),
               jnp.broadcast_to(global_bias, (L,)),
               user_ids.astype(jnp.int32), item_ids.astype(jnp.int32))
